# baseline (speedup 1.0000x reference)
import jax
import jax.numpy as jnp
from jax.experimental import pallas as pl
from jax.experimental.pallas import tpu as pltpu
from jax.experimental.pallas import tpu_sc as plsc

VOCAB = 100000
EMB = 64
PAIR = 2 * EMB
HID = 2048
GATES = 4 * HID
L = 20
B = 32
N_TOK = L * B

_GATHER_WINDOW = 40


def _sc_gather(table_pairs, idx_2d):
    mesh = plsc.VectorSubcoreMesh(core_axis_name="c", subcore_axis_name="s")

    @pl.kernel(
        out_type=jax.ShapeDtypeStruct((N_TOK, PAIR), table_pairs.dtype),
        mesh=mesh,
    )
    def gather_kernel(tbl_hbm, i_hbm, o_hbm):
        def body(i_vmem, o_vmem):
            pltpu.sync_copy(tbl_hbm.at[i_vmem.at[0]], o_vmem)

        pltpu.emit_pipeline(
            body,
            grid=(N_TOK // _GATHER_WINDOW,),
            in_specs=[pl.BlockSpec((1, _GATHER_WINDOW), index_map=lambda i: (i, 0))],
            out_specs=[pl.BlockSpec((_GATHER_WINDOW, PAIR), index_map=lambda i: (i, 0))],
            core_axis_name="s",
            dimension_semantics=(pltpu.PARALLEL,),
        )(i_hbm, o_hbm)

    return gather_kernel(table_pairs, idx_2d)


_N_CHUNK = 16
_CHUNK = GATES // _N_CHUNK

_DOT_T = (((1,), (1,)), ((), ()))


def _lstm_body(pair_ref, mask_ref, wih2_ref, whh_ref, b_ref,
               outs_ref, h_ref, c_ref,
               whh_bf, wih_bf, h_sc, c_sc):
    g = pl.program_id(0)

    @pl.when(g == 0)
    def _init():
        wih_bf[...] = wih2_ref[...].astype(jnp.bfloat16)
        h_sc[...] = jnp.zeros((B, HID), jnp.float32)
        c_sc[...] = jnp.zeros((B, HID), jnp.float32)

    @pl.when(g < _N_CHUNK)
    def _cast_chunk():
        whh_bf[pl.ds(g * _CHUNK, _CHUNK), :] = whh_ref[...].astype(jnp.bfloat16)

    @pl.when(g == _N_CHUNK)
    def _run():
        bias = b_ref[...]

        def step(t, _):
            rows = pl.ds(t * B, B)
            x = (pair_ref[rows, :] * mask_ref[rows, :]).astype(jnp.bfloat16)
            h_bf = h_sc[...].astype(jnp.bfloat16)
            gates = (
                jax.lax.dot_general(x, wih_bf[...], _DOT_T,
                                    preferred_element_type=jnp.float32)
                + jax.lax.dot_general(h_bf, whh_bf[...], _DOT_T,
                                      preferred_element_type=jnp.float32)
                + bias
            )
            i_g = jax.nn.sigmoid(gates[:, 0:HID])
            f_g = jax.nn.sigmoid(gates[:, HID:2 * HID])
            g_g = jnp.tanh(gates[:, 2 * HID:3 * HID])
            o_g = jax.nn.sigmoid(gates[:, 3 * HID:4 * HID])
            c_new = f_g * c_sc[...] + i_g * g_g
            h_new = o_g * jnp.tanh(c_new)
            c_sc[...] = c_new
            h_sc[...] = h_new
            outs_ref[rows, :] = h_new
            return 0

        jax.lax.fori_loop(0, L, step, 0)
        h_ref[...] = h_sc[...]
        c_ref[...] = c_sc[...]


def _lstm(pair, mask, W_ih2, W_hh, b2):
    out_shapes = [
        jax.ShapeDtypeStruct((N_TOK, HID), jnp.float32),
        jax.ShapeDtypeStruct((B, HID), jnp.float32),
        jax.ShapeDtypeStruct((B, HID), jnp.float32),
    ]
    grid = (_N_CHUNK + 1,)
    return pl.pallas_call(
        _lstm_body,
        grid=grid,
        in_specs=[
            pl.BlockSpec((N_TOK, PAIR), lambda g: (0, 0)),
            pl.BlockSpec((N_TOK, PAIR), lambda g: (0, 0)),
            pl.BlockSpec((GATES, PAIR), lambda g: (0, 0)),
            pl.BlockSpec((_CHUNK, HID), lambda g: (jnp.minimum(g, _N_CHUNK - 1), 0)),
            pl.BlockSpec((1, GATES), lambda g: (0, 0)),
        ],
        out_specs=[
            pl.BlockSpec((N_TOK, HID), lambda g: (0, 0)),
            pl.BlockSpec((B, HID), lambda g: (0, 0)),
            pl.BlockSpec((B, HID), lambda g: (0, 0)),
        ],
        out_shape=out_shapes,
        scratch_shapes=[
            pltpu.VMEM((GATES, HID), jnp.bfloat16),
            pltpu.VMEM((GATES, PAIR), jnp.bfloat16),
            pltpu.VMEM((B, HID), jnp.float32),
            pltpu.VMEM((B, HID), jnp.float32),
        ],
    )(pair, mask, W_ih2, W_hh, b2)


_TV = 1024
_N_TILE = (VOCAB + _TV - 1) // _TV

_VSPLIT = 8
_TVS = _TV // _VSPLIT


_RING = 4
_N_FULL = VOCAB // _TV


def _proj_body(outs_ref, *refs):
    w_refs = refs[:_VSPLIT]
    b_ref = refs[_VSPLIT]
    o_hbm = refs[_VSPLIT + 1]
    obuf = refs[_VSPLIT + 2]
    sems = refs[_VSPLIT + 3]
    v = pl.program_id(0)
    slot = jax.lax.rem(v, _RING)

    def copy_full(i):
        sl = jax.lax.rem(i, _RING)
        return pltpu.make_async_copy(
            obuf.at[sl], o_hbm.at[:, pl.ds(i * _TV, _TV)], sems.at[sl])

    @pl.when(v >= _RING)
    def _wait_prev():
        copy_full(v - _RING).wait()

    outs = outs_ref[...]
    for j, w_ref in enumerate(w_refs):
        cols = slice(j * _TVS, (j + 1) * _TVS)
        obuf[slot, :, cols] = (
            jax.lax.dot_general(outs, w_ref[...].astype(jnp.bfloat16),
                                _DOT_T, preferred_element_type=jnp.float32)
            + b_ref[0][:, cols]
        )

    copy_full(v).start()

    @pl.when(v == _N_FULL - 1)
    def _finish():
        for d in range(_RING - 1, -1, -1):
            copy_full(v - d).wait()


def _proj_tail_body(outs_ref, w_ref, b_ref, _logits_in, o_ref):
    o_ref[...] = (
        jax.lax.dot_general(outs_ref[...], w_ref[...].astype(jnp.bfloat16),
                            _DOT_T, preferred_element_type=jnp.float32)
        + b_ref[0]
    )


def _proj(outs_bf, W_lin, b_pad):
    max_blk = (VOCAB + _TVS - 1) // _TVS - 1
    w_spec = [
        pl.BlockSpec((_TVS, HID),
                     lambda v, j=j: (_VSPLIT * v + j, 0))
        for j in range(_VSPLIT)
    ]
    main = pl.pallas_call(
        _proj_body,
        grid=(_N_FULL,),
        in_specs=[
            pl.BlockSpec((N_TOK, HID), lambda v: (0, 0)),
            *w_spec,
            pl.BlockSpec((1, 1, _TV), lambda v: (v, 0, 0)),
        ],
        out_specs=pl.BlockSpec(memory_space=pl.ANY),
        out_shape=jax.ShapeDtypeStruct((N_TOK, VOCAB), jnp.float32),
        scratch_shapes=[
            pltpu.VMEM((_RING, N_TOK, _TV), jnp.float32),
            pltpu.SemaphoreType.DMA((_RING,)),
        ],
    )(outs_bf, *([W_lin] * _VSPLIT), b_pad)

    tail = pl.pallas_call(
        _proj_tail_body,
        grid=(1,),
        in_specs=[
            pl.BlockSpec((N_TOK, HID), lambda v: (0, 0)),
            pl.BlockSpec((_TV, HID), lambda v: (_N_FULL, 0)),
            pl.BlockSpec((1, 1, _TV), lambda v: (_N_FULL, 0, 0)),
            pl.BlockSpec(memory_space=pl.ANY),
        ],
        out_specs=pl.BlockSpec((N_TOK, _TV), lambda v: (0, _N_FULL)),
        out_shape=jax.ShapeDtypeStruct((N_TOK, VOCAB), jnp.float32),
        input_output_aliases={3: 0},
    )(outs_bf, W_lin, b_pad, main)
    return tail


def kernel(x, emb_table, W_ih, W_hh, b_ih, b_hh, W_lin, b_lin):
    ids = x.reshape(N_TOK).astype(jnp.int32)
    idx_2d = (ids // 2).reshape(N_TOK // _GATHER_WINDOW, _GATHER_WINDOW)
    table_pairs = emb_table.reshape(VOCAB // 2, PAIR)
    pair = _sc_gather(table_pairs, idx_2d)

    par = (ids % 2).astype(jnp.float32)[:, None]
    lane = jax.lax.broadcasted_iota(jnp.int32, (1, PAIR), 1)
    mask = jnp.where(lane < EMB, 1.0 - par, par)

    W_ih2 = jnp.concatenate([W_ih, W_ih], axis=1)
    b2 = (b_ih + b_hh).reshape(1, GATES)
    outs, h, c = _lstm(pair, mask, W_ih2, W_hh, b2)

    b_pad = jnp.pad(b_lin, (0, _N_TILE * _TV - VOCAB)).reshape(_N_TILE, 1, _TV)
    logits = _proj(outs.astype(jnp.bfloat16), W_lin, b_pad)
    return logits, h[None], c[None]


# speedup vs baseline: 1.2652x; 1.2652x over previous
import jax
import jax.numpy as jnp
from jax.experimental import pallas as pl
from jax.experimental.pallas import tpu as pltpu
from jax.experimental.pallas import tpu_sc as plsc

VOCAB = 100000
EMB = 64
PAIR = 2 * EMB
HID = 2048
GATES = 4 * HID
L = 20
B = 32
N_TOK = L * B

_GATHER_WINDOW = 40


def _sc_gather(table_pairs, idx_2d):
    mesh = plsc.VectorSubcoreMesh(core_axis_name="c", subcore_axis_name="s")

    @pl.kernel(
        out_type=jax.ShapeDtypeStruct((N_TOK, PAIR), table_pairs.dtype),
        mesh=mesh,
    )
    def gather_kernel(tbl_hbm, i_hbm, o_hbm):
        def body(i_vmem, o_vmem):
            pltpu.sync_copy(tbl_hbm.at[i_vmem.at[0]], o_vmem)

        pltpu.emit_pipeline(
            body,
            grid=(N_TOK // _GATHER_WINDOW,),
            in_specs=[pl.BlockSpec((1, _GATHER_WINDOW), index_map=lambda i: (i, 0))],
            out_specs=[pl.BlockSpec((_GATHER_WINDOW, PAIR), index_map=lambda i: (i, 0))],
            core_axis_name="s",
            dimension_semantics=(pltpu.PARALLEL,),
        )(i_hbm, o_hbm)

    return gather_kernel(table_pairs, idx_2d)


_N_CHUNK = 16
_CHUNK = GATES // _N_CHUNK

_DOT_T = (((1,), (1,)), ((), ()))


def _lstm_body(pair_ref, mask_ref, wih2_ref, whh_ref, b_ref,
               outs_ref, h_ref, c_ref,
               whh_bf, wih_bf, h_sc, c_sc):
    g = pl.program_id(0)

    @pl.when(g == 0)
    def _init():
        wih_bf[...] = wih2_ref[...].astype(jnp.bfloat16)
        h_sc[...] = jnp.zeros((B, HID), jnp.float32)
        c_sc[...] = jnp.zeros((B, HID), jnp.float32)

    @pl.when(g < _N_CHUNK)
    def _cast_chunk():
        whh_bf[pl.ds(g * _CHUNK, _CHUNK), :] = whh_ref[...].astype(jnp.bfloat16)

    @pl.when(g == _N_CHUNK)
    def _run():
        bias = b_ref[...]

        def step(t, _):
            rows = pl.ds(t * B, B)
            x = (pair_ref[rows, :] * mask_ref[rows, :]).astype(jnp.bfloat16)
            h_bf = h_sc[...].astype(jnp.bfloat16)
            gates = (
                jax.lax.dot_general(x, wih_bf[...], _DOT_T,
                                    preferred_element_type=jnp.float32)
                + jax.lax.dot_general(h_bf, whh_bf[...], _DOT_T,
                                      preferred_element_type=jnp.float32)
                + bias
            )
            i_g = jax.nn.sigmoid(gates[:, 0:HID])
            f_g = jax.nn.sigmoid(gates[:, HID:2 * HID])
            g_g = jnp.tanh(gates[:, 2 * HID:3 * HID])
            o_g = jax.nn.sigmoid(gates[:, 3 * HID:4 * HID])
            c_new = f_g * c_sc[...] + i_g * g_g
            h_new = o_g * jnp.tanh(c_new)
            c_sc[...] = c_new
            h_sc[...] = h_new
            outs_ref[rows, :] = h_new
            return 0

        jax.lax.fori_loop(0, L, step, 0)
        h_ref[...] = h_sc[...]
        c_ref[...] = c_sc[...]


def _lstm(pair, mask, W_ih2, W_hh, b2):
    out_shapes = [
        jax.ShapeDtypeStruct((N_TOK, HID), jnp.float32),
        jax.ShapeDtypeStruct((B, HID), jnp.float32),
        jax.ShapeDtypeStruct((B, HID), jnp.float32),
    ]
    grid = (_N_CHUNK + 1,)
    return pl.pallas_call(
        _lstm_body,
        grid=grid,
        in_specs=[
            pl.BlockSpec((N_TOK, PAIR), lambda g: (0, 0)),
            pl.BlockSpec((N_TOK, PAIR), lambda g: (0, 0)),
            pl.BlockSpec((GATES, PAIR), lambda g: (0, 0)),
            pl.BlockSpec((_CHUNK, HID), lambda g: (jnp.minimum(g, _N_CHUNK - 1), 0)),
            pl.BlockSpec((1, GATES), lambda g: (0, 0)),
        ],
        out_specs=[
            pl.BlockSpec((N_TOK, HID), lambda g: (0, 0)),
            pl.BlockSpec((B, HID), lambda g: (0, 0)),
            pl.BlockSpec((B, HID), lambda g: (0, 0)),
        ],
        out_shape=out_shapes,
        scratch_shapes=[
            pltpu.VMEM((GATES, HID), jnp.bfloat16),
            pltpu.VMEM((GATES, PAIR), jnp.bfloat16),
            pltpu.VMEM((B, HID), jnp.float32),
            pltpu.VMEM((B, HID), jnp.float32),
        ],
    )(pair, mask, W_ih2, W_hh, b2)


_TV = 2048
_N_TILE = (VOCAB + _TV - 1) // _TV

_VSPLIT = 8
_TVS = _TV // _VSPLIT


def _proj_body(outs_ref, *refs):
    w_refs = refs[:_VSPLIT]
    b_ref = refs[_VSPLIT]
    o_ref = refs[_VSPLIT + 1]
    outs = outs_ref[...]
    bias = b_ref[0]
    for j, w_ref in enumerate(w_refs):
        cols = slice(j * _TVS, (j + 1) * _TVS)
        o_ref[:, cols] = (
            jax.lax.dot_general(outs, w_ref[...].astype(jnp.bfloat16),
                                _DOT_T, preferred_element_type=jnp.float32)
            + bias[:, cols]
        )


def _proj(outs_bf, W_lin, b_pad):
    max_blk = (VOCAB + _TVS - 1) // _TVS - 1
    w_spec = [
        pl.BlockSpec((_TVS, HID),
                     lambda v, j=j: (jnp.minimum(_VSPLIT * v + j, max_blk), 0))
        for j in range(_VSPLIT)
    ]
    return pl.pallas_call(
        _proj_body,
        grid=(_N_TILE,),
        in_specs=[
            pl.BlockSpec((N_TOK, HID), lambda v: (0, 0)),
            *w_spec,
            pl.BlockSpec((1, 1, _TV), lambda v: (v, 0, 0)),
        ],
        out_specs=pl.BlockSpec((N_TOK, _TV), lambda v: (0, v)),
        out_shape=jax.ShapeDtypeStruct((N_TOK, VOCAB), jnp.float32),
        compiler_params=pltpu.CompilerParams(
            dimension_semantics=("parallel",)),
    )(outs_bf, *([W_lin] * _VSPLIT), b_pad)


def kernel(x, emb_table, W_ih, W_hh, b_ih, b_hh, W_lin, b_lin):
    ids = x.reshape(N_TOK).astype(jnp.int32)
    idx_2d = (ids // 2).reshape(N_TOK // _GATHER_WINDOW, _GATHER_WINDOW)
    table_pairs = emb_table.reshape(VOCAB // 2, PAIR)
    pair = _sc_gather(table_pairs, idx_2d)

    par = (ids % 2).astype(jnp.float32)[:, None]
    lane = jax.lax.broadcasted_iota(jnp.int32, (1, PAIR), 1)
    mask = jnp.where(lane < EMB, 1.0 - par, par)

    W_ih2 = jnp.concatenate([W_ih, W_ih], axis=1)
    b2 = (b_ih + b_hh).reshape(1, GATES)
    outs, h, c = _lstm(pair, mask, W_ih2, W_hh, b2)

    b_pad = jnp.pad(b_lin, (0, _N_TILE * _TV - VOCAB)).reshape(_N_TILE, 1, _TV)
    logits = _proj(outs.astype(jnp.bfloat16), W_lin, b_pad)
    return logits, h[None], c[None]
